# Initial kernel scaffold; baseline (speedup 1.0000x reference)
import jax
import jax.numpy as jnp
from jax.experimental import pallas as pl


def kernel(x, edge_index, batch, W_emb, b_emb, W1, b1, W2, b2, bn_g, bn_b, Wg1, bg1, ln_g, ln_b, Wg2, bg2):
    raise NotImplementedError("write your pallas kernel here")


# speedup vs baseline: 5.7582x; 5.7582x over previous
import functools

import jax
import jax.numpy as jnp
from jax import lax
from jax.experimental import pallas as pl
from jax.experimental.pallas import tpu as pltpu
from jax.experimental.pallas import tpu_sc as plsc

N = 10000
E = 320000
D = 128
H = 128
LYR = 5
G = 16

NC = 2
NS = 16
EPT = E // (NC * NS)
CH = 80
NCHUNK = EPT // CH
NP = 10240
RPT = NP // NS
ZR = 128

BLK = 1000
NBLK = N // BLK


def _seg_body(h_hbm, src_hbm, dst_hbm, out_hbm,
              src_all, dst_all, src_v, dst_v, rows_v, zbuf, acc_sh, sem):
    c = lax.axis_index("c")
    s = lax.axis_index("s")

    def _zb(k, carry):
        i = k // 8
        j = k - i * 8
        zbuf[i, pl.ds(j * 16, 16)] = jnp.zeros((16,), jnp.float32)
        return carry

    lax.fori_loop(0, ZR * 8, _zb, 0)
    for r in range(RPT // ZR):
        pltpu.sync_copy(zbuf, acc_sh.at[pl.ds(s * RPT + r * ZR, ZR)])
    plsc.subcore_barrier()

    base = (c * NS + s) * EPT
    pltpu.sync_copy(src_hbm.at[pl.ds(base, EPT)], src_all)
    pltpu.sync_copy(dst_hbm.at[pl.ds(base, EPT)], dst_all)

    def _chunk(k, carry):
        off = k * CH
        for i in range(CH // 16):
            src_v[pl.ds(i * 16, 16)] = src_all[pl.ds(off + i * 16, 16)]
            dst_v[pl.ds(i * 16, 16)] = dst_all[pl.ds(off + i * 16, 16)]
        pltpu.async_copy(h_hbm.at[src_v], rows_v, sem).wait()
        pltpu.sync_copy(rows_v, acc_sh.at[dst_v], add=True)
        return carry

    lax.fori_loop(0, NCHUNK, _chunk, 0)
    plsc.subcore_barrier()

    pltpu.sync_copy(acc_sh.at[pl.ds(s * RPT, RPT)],
                    out_hbm.at[pl.ds(c * NP + s * RPT, RPT)])


@functools.cache
def _get_seg_sum():
    return pl.kernel(
        _seg_body,
        mesh=plsc.VectorSubcoreMesh(core_axis_name="c", subcore_axis_name="s"),
        out_type=jax.ShapeDtypeStruct((NC * NP, D), jnp.float32),
        scratch_types=[
            pltpu.VMEM((EPT,), jnp.int32),
            pltpu.VMEM((EPT,), jnp.int32),
            pltpu.VMEM((CH,), jnp.int32),
            pltpu.VMEM((CH,), jnp.int32),
            pltpu.VMEM((CH, D), jnp.float32),
            pltpu.VMEM((ZR, D), jnp.float32),
            pltpu.VMEM_SHARED((NP, D), jnp.float32),
            pltpu.SemaphoreType.DMA,
        ],
    )


def _emb_body(x_ref, w_ref, b_ref, o_ref):
    o_ref[...] = (jnp.dot(x_ref[...], w_ref[...],
                          preferred_element_type=jnp.float32) + b_ref[...])


def _emb(x, w, b):
    return pl.pallas_call(
        _emb_body,
        grid=(NBLK,),
        in_specs=[
            pl.BlockSpec((BLK, D), lambda i: (i, 0)),
            pl.BlockSpec((D, H), lambda i: (0, 0)),
            pl.BlockSpec((1, H), lambda i: (0, 0)),
        ],
        out_specs=pl.BlockSpec((BLK, H), lambda i: (i, 0)),
        out_shape=jax.ShapeDtypeStruct((N, H), jnp.float32),
    )(x, w, b)


def _p1_body(h_ref, p0_ref, p1_ref, w1_ref, b1_ref, w2_ref, b2_ref,
             u_ref, st_ref, acc):
    i = pl.program_id(0)
    m = h_ref[...] + p0_ref[...] + p1_ref[...]
    t = jnp.maximum(jnp.dot(m, w1_ref[...],
                            preferred_element_type=jnp.float32) + b1_ref[...],
                    0.0)
    u = jnp.dot(t, w2_ref[...], preferred_element_type=jnp.float32) + b2_ref[...]
    u_ref[...] = u

    @pl.when(i == 0)
    def _():
        acc[...] = jnp.zeros_like(acc)

    acc[...] += jnp.sum(u, axis=0, keepdims=True)

    @pl.when(i == pl.num_programs(0) - 1)
    def _():
        st_ref[...] = acc[...]


def _pass1(h, p0, p1, w1, b1, w2, b2):
    return pl.pallas_call(
        _p1_body,
        grid=(NBLK,),
        in_specs=[
            pl.BlockSpec((BLK, H), lambda i: (i, 0)),
            pl.BlockSpec((BLK, H), lambda i: (i, 0)),
            pl.BlockSpec((BLK, H), lambda i: (i, 0)),
            pl.BlockSpec((H, 2 * H), lambda i: (0, 0)),
            pl.BlockSpec((1, 2 * H), lambda i: (0, 0)),
            pl.BlockSpec((2 * H, H), lambda i: (0, 0)),
            pl.BlockSpec((1, H), lambda i: (0, 0)),
        ],
        out_specs=[
            pl.BlockSpec((BLK, H), lambda i: (i, 0)),
            pl.BlockSpec((1, H), lambda i: (0, 0)),
        ],
        out_shape=[
            jax.ShapeDtypeStruct((N, H), jnp.float32),
            jax.ShapeDtypeStruct((1, H), jnp.float32),
        ],
        scratch_shapes=[pltpu.VMEM((1, H), jnp.float32)],
    )(h, p0, p1, w1, b1, w2, b2)


def _pv_body(u_ref, ms_ref, sq_ref, acc):
    i = pl.program_id(0)

    @pl.when(i == 0)
    def _():
        acc[...] = jnp.zeros_like(acc)

    dv = u_ref[...] - ms_ref[...] / N
    acc[...] += jnp.sum(dv * dv, axis=0, keepdims=True)

    @pl.when(i == pl.num_programs(0) - 1)
    def _():
        sq_ref[...] = acc[...]


def _passvar(u, ms):
    return pl.pallas_call(
        _pv_body,
        grid=(NBLK,),
        in_specs=[
            pl.BlockSpec((BLK, H), lambda i: (i, 0)),
            pl.BlockSpec((1, H), lambda i: (0, 0)),
        ],
        out_specs=pl.BlockSpec((1, H), lambda i: (0, 0)),
        out_shape=jax.ShapeDtypeStruct((1, H), jnp.float32),
        scratch_shapes=[pltpu.VMEM((1, H), jnp.float32)],
    )(u, ms)


def _p2_body(u_ref, ms_ref, sq_ref, hp_ref, g_ref, b_ref, o_ref, *, resid):
    mu = ms_ref[...] / N
    var = sq_ref[...] / N
    y = (u_ref[...] - mu) / jnp.sqrt(var + 1e-5) * g_ref[...] + b_ref[...]
    y = jnp.maximum(y, 0.0)
    if resid:
        y = y + hp_ref[...]
    o_ref[...] = y


def _pass2(u, ms, sq, hprev, g, b, resid):
    return pl.pallas_call(
        functools.partial(_p2_body, resid=resid),
        grid=(NBLK,),
        in_specs=[
            pl.BlockSpec((BLK, H), lambda i: (i, 0)),
            pl.BlockSpec((1, H), lambda i: (0, 0)),
            pl.BlockSpec((1, H), lambda i: (0, 0)),
            pl.BlockSpec((BLK, H), lambda i: (i, 0)),
            pl.BlockSpec((1, H), lambda i: (0, 0)),
            pl.BlockSpec((1, H), lambda i: (0, 0)),
        ],
        out_specs=pl.BlockSpec((BLK, H), lambda i: (i, 0)),
        out_shape=jax.ShapeDtypeStruct((N, H), jnp.float32),
    )(u, ms, sq, hprev, g, b)


def _pool_body(b3_ref, h_ref, wg1_ref, bg1_ref, lng_ref, lnb_ref,
               wg2_ref, bg2_ref, o_ref, sums, cnts):
    i = pl.program_id(0)

    @pl.when(i == 0)
    def _():
        sums[...] = jnp.zeros_like(sums)
        cnts[...] = jnp.zeros_like(cnts)

    bidx = b3_ref[0, 0, :]
    onehot = (bidx[:, None] ==
              lax.broadcasted_iota(jnp.int32, (BLK, G), 1)).astype(jnp.float32)
    sums[...] += lax.dot_general(onehot, h_ref[...],
                                 (((0,), (0,)), ((), ())),
                                 preferred_element_type=jnp.float32)
    cnts[...] += jnp.sum(onehot, axis=0)[:, None]

    @pl.when(i == pl.num_programs(0) - 1)
    def _():
        gf = sums[...] / jnp.maximum(cnts[...], 1.0)
        y = jnp.dot(gf, wg1_ref[...],
                    preferred_element_type=jnp.float32) + bg1_ref[...]
        mu = jnp.mean(y, axis=-1, keepdims=True)
        var = jnp.mean((y - mu) * (y - mu), axis=-1, keepdims=True)
        y = (y - mu) / jnp.sqrt(var + 1e-5) * lng_ref[...] + lnb_ref[...]
        y = jnp.maximum(y, 0.0)
        o_ref[...] = jnp.dot(y, wg2_ref[...],
                             preferred_element_type=jnp.float32) + bg2_ref[...]


def _pool(batch3, h, wg1, bg1, lng, lnb, wg2, bg2):
    return pl.pallas_call(
        _pool_body,
        grid=(NBLK,),
        in_specs=[
            pl.BlockSpec((1, 1, BLK), lambda i: (i, 0, 0)),
            pl.BlockSpec((BLK, H), lambda i: (i, 0)),
            pl.BlockSpec((H, H), lambda i: (0, 0)),
            pl.BlockSpec((1, H), lambda i: (0, 0)),
            pl.BlockSpec((1, H), lambda i: (0, 0)),
            pl.BlockSpec((1, H), lambda i: (0, 0)),
            pl.BlockSpec((H, H), lambda i: (0, 0)),
            pl.BlockSpec((1, H), lambda i: (0, 0)),
        ],
        out_specs=pl.BlockSpec((G, H), lambda i: (0, 0)),
        out_shape=jax.ShapeDtypeStruct((G, H), jnp.float32),
        scratch_shapes=[
            pltpu.VMEM((G, H), jnp.float32),
            pltpu.VMEM((G, 1), jnp.float32),
        ],
    )(batch3, h, wg1, bg1, lng, lnb, wg2, bg2)


def kernel(x, edge_index, batch, W_emb, b_emb, W1, b1, W2, b2,
           bn_g, bn_b, Wg1, bg1, ln_g, ln_b, Wg2, bg2):
    src = edge_index[0]
    dst = edge_index[1]
    batch3 = batch.reshape(NBLK, 1, BLK)

    h = _emb(x, W_emb, b_emb.reshape(1, H))
    for i in range(LYR):
        parts = _get_seg_sum()(h, src, dst)
        p0 = parts[:N]
        p1 = parts[NP:NP + N]
        u, ms = _pass1(h, p0, p1, W1[i], b1[i].reshape(1, 2 * H),
                       W2[i], b2[i].reshape(1, H))
        sq = _passvar(u, ms)
        h = _pass2(u, ms, sq, h, bn_g[i].reshape(1, H), bn_b[i].reshape(1, H),
                   resid=(i > 0))
    graph_features = _pool(batch3, h, Wg1, bg1.reshape(1, H),
                           ln_g.reshape(1, H), ln_b.reshape(1, H),
                           Wg2, bg2.reshape(1, H))
    return (h, graph_features)


# speedup vs baseline: 7.1292x; 1.2381x over previous
import functools

import jax
import jax.numpy as jnp
from jax import lax
from jax.experimental import pallas as pl
from jax.experimental.pallas import tpu as pltpu
from jax.experimental.pallas import tpu_sc as plsc

N = 10000
E = 320000
D = 128
H = 128
LYR = 5
G = 16

NC = 2
NS = 16
EPT = E // (NC * NS)
CH = 80
NCHUNK = EPT // CH
NP = 10240
RPT = NP // NS
ZR = 32

BLK = 1000
NBLK = N // BLK


def _seg_body(h_hbm, src_hbm, dst_hbm, out_hbm,
              src_all, dst_all, src_a, dst_a, src_b, dst_b,
              rows_a, rows_b, zbuf, acc_sh, sem_a, sem_b):
    c = lax.axis_index("c")
    s = lax.axis_index("s")

    def _zb(k, carry):
        i = k // 8
        j = k - i * 8
        zbuf[i, pl.ds(j * 16, 16)] = jnp.zeros((16,), jnp.float32)
        return carry

    lax.fori_loop(0, ZR * 8, _zb, 0)

    def _zc(r, carry):
        pltpu.sync_copy(zbuf, acc_sh.at[pl.ds(s * RPT + r * ZR, ZR)])
        return carry

    lax.fori_loop(0, RPT // ZR, _zc, 0)
    plsc.subcore_barrier()

    base = (c * NS + s) * EPT
    pltpu.sync_copy(src_hbm.at[pl.ds(base, EPT)], src_all)
    pltpu.sync_copy(dst_hbm.at[pl.ds(base, EPT)], dst_all)

    def _stage(k, sv, dv):
        off = k * CH
        for i in range(CH // 16):
            sv[pl.ds(i * 16, 16)] = src_all[pl.ds(off + i * 16, 16)]
            dv[pl.ds(i * 16, 16)] = dst_all[pl.ds(off + i * 16, 16)]

    def _pair(p, carry):
        k0 = p * 2
        _stage(k0, src_a, dst_a)
        ca = pltpu.async_copy(h_hbm.at[src_a], rows_a, sem_a)
        _stage(k0 + 1, src_b, dst_b)
        cb = pltpu.async_copy(h_hbm.at[src_b], rows_b, sem_b)
        ca.wait()
        pltpu.sync_copy(rows_a, acc_sh.at[dst_a], add=True)
        cb.wait()
        pltpu.sync_copy(rows_b, acc_sh.at[dst_b], add=True)
        return carry

    lax.fori_loop(0, NCHUNK // 2, _pair, 0)
    _stage(NCHUNK - 1, src_a, dst_a)
    pltpu.async_copy(h_hbm.at[src_a], rows_a, sem_a).wait()
    pltpu.sync_copy(rows_a, acc_sh.at[dst_a], add=True)
    plsc.subcore_barrier()

    pltpu.sync_copy(acc_sh.at[pl.ds(s * RPT, RPT)],
                    out_hbm.at[pl.ds(c * NP + s * RPT, RPT)])


@functools.cache
def _get_seg_sum():
    return pl.kernel(
        _seg_body,
        mesh=plsc.VectorSubcoreMesh(core_axis_name="c", subcore_axis_name="s"),
        out_type=jax.ShapeDtypeStruct((NC * NP, D), jnp.float32),
        scratch_types=[
            pltpu.VMEM((EPT,), jnp.int32),
            pltpu.VMEM((EPT,), jnp.int32),
            pltpu.VMEM((CH,), jnp.int32),
            pltpu.VMEM((CH,), jnp.int32),
            pltpu.VMEM((CH,), jnp.int32),
            pltpu.VMEM((CH,), jnp.int32),
            pltpu.VMEM((CH, D), jnp.float32),
            pltpu.VMEM((CH, D), jnp.float32),
            pltpu.VMEM((ZR, D), jnp.float32),
            pltpu.VMEM_SHARED((NP, D), jnp.float32),
            pltpu.SemaphoreType.DMA,
            pltpu.SemaphoreType.DMA,
        ],
    )


def _emb_body(x_ref, w_ref, b_ref, o_ref):
    o_ref[...] = (jnp.dot(x_ref[...], w_ref[...],
                          preferred_element_type=jnp.float32) + b_ref[...])


def _emb(x, w, b):
    return pl.pallas_call(
        _emb_body,
        grid=(NBLK,),
        in_specs=[
            pl.BlockSpec((BLK, D), lambda i: (i, 0)),
            pl.BlockSpec((D, H), lambda i: (0, 0)),
            pl.BlockSpec((1, H), lambda i: (0, 0)),
        ],
        out_specs=pl.BlockSpec((BLK, H), lambda i: (i, 0)),
        out_shape=jax.ShapeDtypeStruct((N, H), jnp.float32),
    )(x, w, b)


def _p1_body(h_ref, p0_ref, p1_ref, w1_ref, b1_ref, w2_ref, b2_ref,
             u_ref, st_ref, acc):
    i = pl.program_id(0)
    m = h_ref[...] + p0_ref[...] + p1_ref[...]
    t = jnp.maximum(jnp.dot(m, w1_ref[...],
                            preferred_element_type=jnp.float32) + b1_ref[...],
                    0.0)
    u = jnp.dot(t, w2_ref[...], preferred_element_type=jnp.float32) + b2_ref[...]
    u_ref[...] = u

    @pl.when(i == 0)
    def _():
        acc[...] = jnp.zeros_like(acc)

    acc[...] += jnp.sum(u, axis=0, keepdims=True)

    @pl.when(i == pl.num_programs(0) - 1)
    def _():
        st_ref[...] = acc[...]


def _pass1(h, p0, p1, w1, b1, w2, b2):
    return pl.pallas_call(
        _p1_body,
        grid=(NBLK,),
        in_specs=[
            pl.BlockSpec((BLK, H), lambda i: (i, 0)),
            pl.BlockSpec((BLK, H), lambda i: (i, 0)),
            pl.BlockSpec((BLK, H), lambda i: (i, 0)),
            pl.BlockSpec((H, 2 * H), lambda i: (0, 0)),
            pl.BlockSpec((1, 2 * H), lambda i: (0, 0)),
            pl.BlockSpec((2 * H, H), lambda i: (0, 0)),
            pl.BlockSpec((1, H), lambda i: (0, 0)),
        ],
        out_specs=[
            pl.BlockSpec((BLK, H), lambda i: (i, 0)),
            pl.BlockSpec((1, H), lambda i: (0, 0)),
        ],
        out_shape=[
            jax.ShapeDtypeStruct((N, H), jnp.float32),
            jax.ShapeDtypeStruct((1, H), jnp.float32),
        ],
        scratch_shapes=[pltpu.VMEM((1, H), jnp.float32)],
    )(h, p0, p1, w1, b1, w2, b2)


def _pv_body(u_ref, ms_ref, sq_ref, acc):
    i = pl.program_id(0)

    @pl.when(i == 0)
    def _():
        acc[...] = jnp.zeros_like(acc)

    dv = u_ref[...] - ms_ref[...] / N
    acc[...] += jnp.sum(dv * dv, axis=0, keepdims=True)

    @pl.when(i == pl.num_programs(0) - 1)
    def _():
        sq_ref[...] = acc[...]


def _passvar(u, ms):
    return pl.pallas_call(
        _pv_body,
        grid=(NBLK,),
        in_specs=[
            pl.BlockSpec((BLK, H), lambda i: (i, 0)),
            pl.BlockSpec((1, H), lambda i: (0, 0)),
        ],
        out_specs=pl.BlockSpec((1, H), lambda i: (0, 0)),
        out_shape=jax.ShapeDtypeStruct((1, H), jnp.float32),
        scratch_shapes=[pltpu.VMEM((1, H), jnp.float32)],
    )(u, ms)


def _p2_body(u_ref, ms_ref, sq_ref, hp_ref, g_ref, b_ref, o_ref, *, resid):
    mu = ms_ref[...] / N
    var = sq_ref[...] / N
    y = (u_ref[...] - mu) / jnp.sqrt(var + 1e-5) * g_ref[...] + b_ref[...]
    y = jnp.maximum(y, 0.0)
    if resid:
        y = y + hp_ref[...]
    o_ref[...] = y


def _pass2(u, ms, sq, hprev, g, b, resid):
    return pl.pallas_call(
        functools.partial(_p2_body, resid=resid),
        grid=(NBLK,),
        in_specs=[
            pl.BlockSpec((BLK, H), lambda i: (i, 0)),
            pl.BlockSpec((1, H), lambda i: (0, 0)),
            pl.BlockSpec((1, H), lambda i: (0, 0)),
            pl.BlockSpec((BLK, H), lambda i: (i, 0)),
            pl.BlockSpec((1, H), lambda i: (0, 0)),
            pl.BlockSpec((1, H), lambda i: (0, 0)),
        ],
        out_specs=pl.BlockSpec((BLK, H), lambda i: (i, 0)),
        out_shape=jax.ShapeDtypeStruct((N, H), jnp.float32),
    )(u, ms, sq, hprev, g, b)


def _pool_body(b3_ref, h_ref, wg1_ref, bg1_ref, lng_ref, lnb_ref,
               wg2_ref, bg2_ref, o_ref, sums, cnts):
    i = pl.program_id(0)

    @pl.when(i == 0)
    def _():
        sums[...] = jnp.zeros_like(sums)
        cnts[...] = jnp.zeros_like(cnts)

    bidx = b3_ref[0, 0, :]
    onehot = (bidx[:, None] ==
              lax.broadcasted_iota(jnp.int32, (BLK, G), 1)).astype(jnp.float32)
    sums[...] += lax.dot_general(onehot, h_ref[...],
                                 (((0,), (0,)), ((), ())),
                                 preferred_element_type=jnp.float32)
    cnts[...] += jnp.sum(onehot, axis=0)[:, None]

    @pl.when(i == pl.num_programs(0) - 1)
    def _():
        gf = sums[...] / jnp.maximum(cnts[...], 1.0)
        y = jnp.dot(gf, wg1_ref[...],
                    preferred_element_type=jnp.float32) + bg1_ref[...]
        mu = jnp.mean(y, axis=-1, keepdims=True)
        var = jnp.mean((y - mu) * (y - mu), axis=-1, keepdims=True)
        y = (y - mu) / jnp.sqrt(var + 1e-5) * lng_ref[...] + lnb_ref[...]
        y = jnp.maximum(y, 0.0)
        o_ref[...] = jnp.dot(y, wg2_ref[...],
                             preferred_element_type=jnp.float32) + bg2_ref[...]


def _pool(batch3, h, wg1, bg1, lng, lnb, wg2, bg2):
    return pl.pallas_call(
        _pool_body,
        grid=(NBLK,),
        in_specs=[
            pl.BlockSpec((1, 1, BLK), lambda i: (i, 0, 0)),
            pl.BlockSpec((BLK, H), lambda i: (i, 0)),
            pl.BlockSpec((H, H), lambda i: (0, 0)),
            pl.BlockSpec((1, H), lambda i: (0, 0)),
            pl.BlockSpec((1, H), lambda i: (0, 0)),
            pl.BlockSpec((1, H), lambda i: (0, 0)),
            pl.BlockSpec((H, H), lambda i: (0, 0)),
            pl.BlockSpec((1, H), lambda i: (0, 0)),
        ],
        out_specs=pl.BlockSpec((G, H), lambda i: (0, 0)),
        out_shape=jax.ShapeDtypeStruct((G, H), jnp.float32),
        scratch_shapes=[
            pltpu.VMEM((G, H), jnp.float32),
            pltpu.VMEM((G, 1), jnp.float32),
        ],
    )(batch3, h, wg1, bg1, lng, lnb, wg2, bg2)


def kernel(x, edge_index, batch, W_emb, b_emb, W1, b1, W2, b2,
           bn_g, bn_b, Wg1, bg1, ln_g, ln_b, Wg2, bg2):
    src = edge_index[0]
    dst = edge_index[1]
    batch3 = batch.reshape(NBLK, 1, BLK)

    h = _emb(x, W_emb, b_emb.reshape(1, H))
    for i in range(LYR):
        parts = _get_seg_sum()(h, src, dst)
        p0 = parts[:N]
        p1 = parts[NP:NP + N]
        u, ms = _pass1(h, p0, p1, W1[i], b1[i].reshape(1, 2 * H),
                       W2[i], b2[i].reshape(1, H))
        sq = _passvar(u, ms)
        h = _pass2(u, ms, sq, h, bn_g[i].reshape(1, H), bn_b[i].reshape(1, H),
                   resid=(i > 0))
    graph_features = _pool(batch3, h, Wg1, bg1.reshape(1, H),
                           ln_g.reshape(1, H), ln_b.reshape(1, H),
                           Wg2, bg2.reshape(1, H))
    return (h, graph_features)
